# Initial kernel scaffold; baseline (speedup 1.0000x reference)
import jax
import jax.numpy as jnp
from jax.experimental import pallas as pl


def kernel(exercise_tokens, category_tokens, exercise_table, category_table, position_table):
    raise NotImplementedError("write your pallas kernel here")


# speedup vs baseline: 6.0775x; 6.0775x over previous
import functools

import jax
import jax.numpy as jnp
from jax import lax
from jax.experimental import pallas as pl
from jax.experimental.pallas import tpu as pltpu
from jax.experimental.pallas import tpu_sc as plsc

DIM = 64
ROWS = 100


@functools.lru_cache(maxsize=None)
def _build(num_blocks):
    info = plsc.get_sparse_core_info()
    nc, ns = info.num_cores, info.num_subcores
    nw = nc * ns
    bpw = num_blocks // nw
    mesh = plsc.VectorSubcoreMesh(core_axis_name="c", subcore_axis_name="s")

    @functools.partial(
        pl.kernel,
        mesh=mesh,
        out_type=jax.ShapeDtypeStruct((num_blocks, ROWS, DIM), jnp.float32),
        compiler_params=pltpu.CompilerParams(use_tc_tiling_on_sc=False),
        scratch_types=[
            pltpu.VMEM((ROWS,), jnp.int32),
            pltpu.VMEM((ROWS,), jnp.int32),
            pltpu.VMEM((ROWS,), jnp.int32),
            pltpu.VMEM((ROWS, DIM), jnp.float32),
            pltpu.VMEM((ROWS, DIM), jnp.float32),
            pltpu.VMEM((2, ROWS, DIM), jnp.float32),
            pltpu.VMEM_SHARED((ns * ROWS, DIM), jnp.float32),
            pltpu.SemaphoreType.DMA,
            pltpu.SemaphoreType.DMA,
        ],
    )
    def k(ex_idx, cat_idx, ident, ex_tab, cat_tab, pos_tab, out,
          exv, catv, identv, exbuf, catbuf, posbuf, slab, sem_a, sem_b):
        cid = lax.axis_index("c")
        sid = lax.axis_index("s")
        wid = sid * nc + cid
        base = wid * bpw
        pltpu.sync_copy(ident.at[sid], identv)
        pltpu.sync_copy(pos_tab, posbuf)

        def body(i, carry):
            r = base + i
            j = lax.rem(i, 2)
            pltpu.sync_copy(ex_idx.at[r], exv)
            pltpu.sync_copy(cat_idx.at[r], catv)
            cp_ex = pltpu.async_copy(ex_tab.at[exv], exbuf, sem_a)
            cp_cat = pltpu.async_copy(cat_tab.at[catv], catbuf, sem_b)
            pltpu.sync_copy(posbuf.at[j], slab.at[pl.ds(sid * ROWS, ROWS)])
            cp_ex.wait()
            cp_cat.wait()
            pltpu.sync_copy(exbuf, slab.at[identv], add=True)
            pltpu.sync_copy(catbuf, slab.at[identv], add=True)
            pltpu.sync_copy(slab.at[pl.ds(sid * ROWS, ROWS)], out.at[r])
            return carry

        lax.fori_loop(0, bpw, body, 0)

    return k


def kernel(exercise_tokens, category_tokens, exercise_table, category_table,
           position_table):
    batch, seq = exercise_tokens.shape
    dim = exercise_table.shape[1]
    num_blocks = (batch * seq) // ROWS
    ex_idx = exercise_tokens.reshape(num_blocks, ROWS).astype(jnp.int32)
    cat_idx = category_tokens.reshape(num_blocks, ROWS).astype(jnp.int32)
    info = plsc.get_sparse_core_info()
    ident = (jnp.arange(ROWS, dtype=jnp.int32)[None, :]
             + ROWS * jnp.arange(info.num_subcores, dtype=jnp.int32)[:, None])
    pos = position_table.reshape(seq // ROWS, ROWS, dim)
    k = _build(num_blocks)
    out = k(ex_idx, cat_idx, ident, exercise_table, category_table, pos)
    return out.reshape(batch, seq, dim)


# speedup vs baseline: 8.6648x; 1.4257x over previous
import functools

import jax
import jax.numpy as jnp
from jax import lax
from jax.experimental import pallas as pl
from jax.experimental.pallas import tpu as pltpu
from jax.experimental.pallas import tpu_sc as plsc

DIM = 64
ROWS = 100
SB = 16


@functools.lru_cache(maxsize=None)
def _build(num_blocks):
    info = plsc.get_sparse_core_info()
    nc, ns = info.num_cores, info.num_subcores
    nw = nc * ns
    bpw = num_blocks // nw
    nsb = bpw // SB
    mesh = plsc.VectorSubcoreMesh(core_axis_name="c", subcore_axis_name="s")

    @functools.partial(
        pl.kernel,
        mesh=mesh,
        out_type=jax.ShapeDtypeStruct((num_blocks, ROWS, DIM), jnp.float32),
        compiler_params=pltpu.CompilerParams(use_tc_tiling_on_sc=False),
        scratch_types=[
            pltpu.VMEM((SB, ROWS), jnp.int32),
            pltpu.VMEM((SB, ROWS), jnp.int32),
            pltpu.VMEM((2, ROWS), jnp.int32),
            pltpu.VMEM((2, ROWS, DIM), jnp.float32),
            pltpu.VMEM((2, ROWS, DIM), jnp.float32),
            pltpu.VMEM((2, ROWS, DIM), jnp.float32),
            pltpu.VMEM_SHARED((ns * 2 * ROWS, DIM), jnp.float32),
            pltpu.SemaphoreType.DMA,
            pltpu.SemaphoreType.DMA,
            pltpu.SemaphoreType.DMA,
            pltpu.SemaphoreType.DMA,
        ],
    )
    def k(ex_idx, cat_idx, ident, ex_tab, cat_tab, pos_tab, out,
          exidx_s, catidx_s, identv, exbuf, catbuf, posbuf, slab,
          sem_g0, sem_g1, sem_w0, sem_w1):
        cid = lax.axis_index("c")
        sid = lax.axis_index("s")
        wid = sid * nc + cid
        base = wid * bpw
        sem_g = (sem_g0, sem_g1)
        sem_w = (sem_w0, sem_w1)
        pltpu.sync_copy(ident.at[sid], identv)
        pltpu.sync_copy(pos_tab, posbuf)

        def slab_slice(c):
            return slab.at[pl.ds((sid * 2 + c) * ROWS, ROWS)]

        def issue_gathers(j, c):
            pltpu.async_copy(ex_tab.at[exidx_s.at[j]], exbuf.at[c], sem_g[c])
            pltpu.async_copy(cat_tab.at[catidx_s.at[j]], catbuf.at[c],
                             sem_g[c])

        def wait_gathers(j, c):
            pltpu.make_async_copy(
                ex_tab.at[exidx_s.at[j]], exbuf.at[c], sem_g[c]).wait()
            pltpu.make_async_copy(
                cat_tab.at[catidx_s.at[j]], catbuf.at[c], sem_g[c]).wait()

        def consume(i, j, c):
            wait_gathers(j, c)

            @pl.when(i >= 2)
            def _():
                pltpu.make_async_copy(
                    slab_slice(c), out.at[base + i], sem_w[c]).wait()

            pltpu.sync_copy(posbuf.at[c], slab_slice(c))
            pltpu.sync_copy(exbuf.at[c], slab.at[identv.at[c]], add=True)
            pltpu.sync_copy(catbuf.at[c], slab.at[identv.at[c]], add=True)
            pltpu.async_copy(slab_slice(c), out.at[base + i], sem_w[c])

        def outer(sb, carry):
            sb_base = base + sb * SB
            pltpu.sync_copy(ex_idx.at[pl.ds(sb_base, SB)], exidx_s)
            pltpu.sync_copy(cat_idx.at[pl.ds(sb_base, SB)], catidx_s)
            issue_gathers(0, 0)

            def inner(g, carry2):
                for half in range(2):
                    j = 2 * g + half
                    c = half
                    i = sb * SB + j

                    @pl.when(j + 1 < SB)
                    def _():
                        issue_gathers(j + 1, 1 - c)

                    consume(i, j, c)
                return carry2

            lax.fori_loop(0, SB // 2, inner, 0)
            return carry

        lax.fori_loop(0, nsb, outer, 0)
        for c in range(2):
            pltpu.make_async_copy(
                slab_slice(c), out.at[base], sem_w[c]).wait()

    return k


def kernel(exercise_tokens, category_tokens, exercise_table, category_table,
           position_table):
    batch, seq = exercise_tokens.shape
    dim = exercise_table.shape[1]
    num_blocks = (batch * seq) // ROWS
    ex_idx = exercise_tokens.reshape(num_blocks, ROWS).astype(jnp.int32)
    cat_idx = category_tokens.reshape(num_blocks, ROWS).astype(jnp.int32)
    info = plsc.get_sparse_core_info()
    ident = (jnp.arange(ROWS, dtype=jnp.int32)[None, :]
             + ROWS * jnp.arange(info.num_subcores * 2,
                                 dtype=jnp.int32)[:, None])
    ident = ident.reshape(info.num_subcores, 2, ROWS)
    pos = position_table.reshape(seq // ROWS, ROWS, dim)
    k = _build(num_blocks)
    out = k(ex_idx, cat_idx, ident, exercise_table, category_table, pos)
    return out.reshape(batch, seq, dim)
